# baseline (speedup 1.0000x reference)
import dataclasses
import functools

import jax
import jax.numpy as jnp
from jax import lax
from jax.experimental import pallas as pl
from jax.experimental.pallas import tpu as pltpu
from jax.experimental.pallas import tpu_sc as plsc

NC = 2
NS = 16
L = 16
NW = NC * NS

ROWS, COLS = 4096, 200
SPLIT = 1024
NSTRIPE = SPLIT // 128
NBAND = NW // NSTRIPE
RB = 56
BC = 512
NSC_BLOCKS = SPLIT // BC


def _sc_lookup_t(inputs_t, mapping):
    mesh = plsc.VectorSubcoreMesh(
        core_axis_name="c", subcore_axis_name="s",
        num_cores=NC, num_subcores=NS)
    cp = pltpu.CompilerParams()
    if "needs_layout_passes" in pltpu.CompilerParams.__dataclass_fields__:
        cp = dataclasses.replace(cp, needs_layout_passes=False,
                                 use_tc_tiling_on_sc=True)

    @functools.partial(
        pl.kernel,
        out_type=jax.ShapeDtypeStruct((COLS, SPLIT), jnp.int32),
        mesh=mesh,
        scratch_types=[
            pltpu.VMEM((128,), jnp.int32),
            pltpu.VMEM((RB, 128), jnp.int32),
            pltpu.VMEM((RB, 128), jnp.int32),
        ],
        compiler_params=cp,
    )
    def lookup_kernel(in_hbm, map_hbm, out_hbm, table_v, idx_v, out_v):
        wid = lax.axis_index("s") * NC + lax.axis_index("c")
        col0 = (wid % NSTRIPE) * 128
        row0 = pl.multiple_of(
            jnp.minimum((wid // NSTRIPE) * RB, COLS - RB), 8)
        pltpu.sync_copy(map_hbm, table_v)
        pltpu.sync_copy(in_hbm.at[pl.ds(row0, RB), pl.ds(col0, 128)], idx_v)

        @plsc.parallel_loop(0, RB, step=1, unroll=2)
        def _(r):
            for o in range(0, 128, L):
                idx = idx_v[r, pl.ds(o, L)]
                out_v[r, pl.ds(o, L)] = plsc.load_gather(table_v, [idx])

        pltpu.sync_copy(out_v, out_hbm.at[pl.ds(row0, RB), pl.ds(col0, 128)])

    return lookup_kernel(inputs_t, mapping)


def _tc_merge_lookup_t(inputs_t, map2d, sc_part):
    def body(in_ref, map_ref, sc_ref, out_ref):
        i = pl.program_id(0)

        @pl.when(i < NSC_BLOCKS)
        def _():
            out_ref[...] = sc_ref[...]

        @pl.when(i >= NSC_BLOCKS)
        def _():
            idx = in_ref[...]
            table = map_ref[...]
            table_b = jnp.broadcast_to(table, (idx.shape[0], 128))
            out_ref[...] = jnp.take_along_axis(table_b, idx, axis=1)

    return pl.pallas_call(
        body,
        out_shape=jax.ShapeDtypeStruct((COLS, ROWS), jnp.int32),
        grid=(ROWS // BC,),
        in_specs=[
            pl.BlockSpec((COLS, BC), lambda i: (0, i)),
            pl.BlockSpec((1, 128), lambda i: (0, 0)),
            pl.BlockSpec((COLS, BC),
                         lambda i: (0, jnp.minimum(i, NSC_BLOCKS - 1))),
        ],
        out_specs=pl.BlockSpec((COLS, BC), lambda i: (0, i)),
    )(inputs_t, map2d, sc_part)


@jax.jit
def _lookup(inputs, mapping):
    t = inputs.T
    map2d = mapping.reshape(1, 128)
    sc_part = _sc_lookup_t(t, mapping)
    return _tc_merge_lookup_t(t, map2d, sc_part).T


def kernel(inputs, mapping):
    return _lookup(inputs, mapping)


# speedup vs baseline: 1.1715x; 1.1715x over previous
import dataclasses
import functools

import jax
import jax.numpy as jnp
from jax import lax
from jax.experimental import pallas as pl
from jax.experimental.pallas import tpu as pltpu
from jax.experimental.pallas import tpu_sc as plsc

NC = 2
NS = 16
L = 16
NW = NC * NS

ROWS, COLS = 4096, 200
CPW = ROWS // NW


@jax.jit
def _sc_lookup_t(inputs_t, mapping):
    mesh = plsc.VectorSubcoreMesh(
        core_axis_name="c", subcore_axis_name="s",
        num_cores=NC, num_subcores=NS)
    cp = pltpu.CompilerParams()
    if "needs_layout_passes" in pltpu.CompilerParams.__dataclass_fields__:
        cp = dataclasses.replace(cp, needs_layout_passes=False,
                                 use_tc_tiling_on_sc=True)

    @functools.partial(
        pl.kernel,
        out_type=jax.ShapeDtypeStruct((COLS, ROWS), jnp.int32),
        mesh=mesh,
        scratch_types=[
            pltpu.VMEM((128,), jnp.int32),
            pltpu.VMEM((COLS, CPW), jnp.int32),
            pltpu.VMEM((COLS, CPW), jnp.int32),
        ],
        compiler_params=cp,
    )
    def lookup_kernel(in_hbm, map_hbm, out_hbm, table_v, idx_v, out_v):
        wid = lax.axis_index("s") * NC + lax.axis_index("c")
        col0 = wid * CPW
        pltpu.sync_copy(map_hbm, table_v)
        pltpu.sync_copy(in_hbm.at[:, pl.ds(col0, CPW)], idx_v)

        @plsc.parallel_loop(0, COLS, step=1, unroll=4)
        def _(r):
            for o in range(0, CPW, L):
                idx = idx_v[r, pl.ds(o, L)]
                out_v[r, pl.ds(o, L)] = plsc.load_gather(table_v, [idx])

        pltpu.sync_copy(out_v, out_hbm.at[:, pl.ds(col0, CPW)])

    return lookup_kernel(inputs_t, mapping)


def kernel(inputs, mapping):
    return _sc_lookup_t(inputs.T, mapping).T


# speedup vs baseline: 1.2212x; 1.0424x over previous
import dataclasses
import functools

import jax
import jax.numpy as jnp
from jax import lax
from jax.experimental import pallas as pl
from jax.experimental.pallas import tpu as pltpu
from jax.experimental.pallas import tpu_sc as plsc

NC = 2
NS = 16
L = 16
NW = NC * NS

ROWS, COLS = 4096, 200
CPW = ROWS // NW


@jax.jit
def _sc_lookup_t(inputs_t, mapping):
    mesh = plsc.VectorSubcoreMesh(
        core_axis_name="c", subcore_axis_name="s",
        num_cores=NC, num_subcores=NS)
    cp = pltpu.CompilerParams()
    if "needs_layout_passes" in pltpu.CompilerParams.__dataclass_fields__:
        cp = dataclasses.replace(cp, needs_layout_passes=False,
                                 use_tc_tiling_on_sc=True)

    @functools.partial(
        pl.kernel,
        out_type=jax.ShapeDtypeStruct((COLS, ROWS), jnp.int32),
        mesh=mesh,
        scratch_types=[
            pltpu.VMEM((128,), jnp.int32),
            pltpu.VMEM((COLS, CPW), jnp.int32),
            pltpu.VMEM((COLS, CPW), jnp.int32),
            pltpu.SemaphoreType.DMA,
            pltpu.SemaphoreType.DMA,
            pltpu.SemaphoreType.DMA,
        ],
        compiler_params=cp,
    )
    def lookup_kernel(in_hbm, map_hbm, out_hbm, table_v, idx_v, out_v,
                      s0, s1, s2):
        wid = lax.axis_index("s") * NC + lax.axis_index("c")
        col0 = wid * CPW
        cin0 = pltpu.async_copy(
            in_hbm.at[pl.ds(0, 104), pl.ds(col0, CPW)],
            idx_v.at[pl.ds(0, 104)], s0)
        cin1 = pltpu.async_copy(
            in_hbm.at[pl.ds(104, 96), pl.ds(col0, CPW)],
            idx_v.at[pl.ds(104, 96)], s1)
        pltpu.sync_copy(map_hbm, table_v)
        cin0.wait()

        @plsc.parallel_loop(0, 104, step=1, unroll=4)
        def _(r):
            for o in range(0, CPW, L):
                idx = idx_v[r, pl.ds(o, L)]
                out_v[r, pl.ds(o, L)] = plsc.load_gather(table_v, [idx])

        cout0 = pltpu.async_copy(
            out_v.at[pl.ds(0, 104)],
            out_hbm.at[pl.ds(0, 104), pl.ds(col0, CPW)], s2)
        cin1.wait()

        @plsc.parallel_loop(104, COLS, step=1, unroll=4)
        def _(r):
            for o in range(0, CPW, L):
                idx = idx_v[r, pl.ds(o, L)]
                out_v[r, pl.ds(o, L)] = plsc.load_gather(table_v, [idx])

        cout0.wait()
        pltpu.sync_copy(out_v.at[pl.ds(104, 96)],
                        out_hbm.at[pl.ds(104, 96), pl.ds(col0, CPW)])

    return lookup_kernel(inputs_t, mapping)


def kernel(inputs, mapping):
    return _sc_lookup_t(inputs.T, mapping).T
